# baseline (speedup 1.0000x reference)
import functools

import jax
import jax.numpy as jnp
from jax import lax
from jax.experimental import pallas as pl
from jax.experimental.pallas import tpu as pltpu
from jax.experimental.pallas import tpu_sc as plsc

B = 4096
H = 128
SEQ = 200
S = SEQ + 2

NC = 1
NS = 16
NW = NC * NS
BPW = (2 * B) // NW
CH = BPW // 128
CW = BPW // CH
LANES = 16

_mesh = plsc.VectorSubcoreMesh(
    core_axis_name="c", subcore_axis_name="s", num_cores=NC
)


@functools.partial(
    pl.kernel,
    out_type=jax.ShapeDtypeStruct((2 * B, H), jnp.float32),
    mesh=_mesh,
    scratch_types=[
        pltpu.VMEM((BPW,), jnp.int32),
        pltpu.VMEM((CH, CW), jnp.int32),
        pltpu.VMEM((BPW, H), jnp.float32),
        pltpu.SemaphoreType.DMA,
    ],
)
def _stack_gather(stack_hbm, pos_hbm, out_hbm, pos_v, idx_v, rows_v, sem):
    wid = lax.axis_index("s") * NC + lax.axis_index("c")
    nwh = NW // 2
    b0 = (wid % nwh) * BPW
    jf = wid // nwh

    pltpu.sync_copy(pos_hbm.at[pl.ds(b0, BPW)], pos_v)

    for t in range(BPW // LANES):
        p = pos_v[pl.ds(t * LANES, LANES)]
        blane = b0 + t * LANES + lax.iota(jnp.int32, LANES)
        pm = p - 1
        pm = jnp.where(pm < 0, pm + S, pm)
        row = pm + (p - pm) * jf
        k = t // (CW // LANES)
        c = (t % (CW // LANES)) * LANES
        idx_v[k, pl.ds(c, LANES)] = row * B + blane

    copies = [
        pltpu.async_copy(
            stack_hbm.at[idx_v.at[k]],
            rows_v.at[pl.ds(k * CW, CW)],
            sem,
        )
        for k in range(CH)
    ]
    for cp in copies:
        cp.wait()

    pltpu.sync_copy(rows_v, out_hbm.at[pl.ds(wid * BPW, BPW)])


def kernel(input, op, hidden_stack, pos):
    del input, op
    stack_flat = hidden_stack.reshape(S * B, H)
    out = _stack_gather(stack_flat, pos)
    return out.reshape(2, B, H)


# speedup vs baseline: 1.0230x; 1.0230x over previous
import functools

import jax
import jax.numpy as jnp
from jax import lax
from jax.experimental import pallas as pl
from jax.experimental.pallas import tpu as pltpu
from jax.experimental.pallas import tpu_sc as plsc

B = 4096
H = 128
SEQ = 200
S = SEQ + 2

NC = 2
NS = 16
NW = NC * NS
BPW = (2 * B) // NW
CH = BPW // 128
CW = BPW // CH
LANES = 16

_mesh = plsc.VectorSubcoreMesh(
    core_axis_name="c", subcore_axis_name="s", num_cores=NC
)


@functools.partial(
    pl.kernel,
    out_type=jax.ShapeDtypeStruct((2 * B, H), jnp.float32),
    mesh=_mesh,
    scratch_types=[
        pltpu.VMEM((BPW,), jnp.int32),
        pltpu.VMEM((CH, CW), jnp.int32),
        pltpu.VMEM((BPW, H), jnp.float32),
        pltpu.SemaphoreType.DMA,
        pltpu.SemaphoreType.DMA,
    ],
)
def _stack_gather(stack_hbm, pos_hbm, out_hbm, pos_v, idx_v, rows_v, gsem, wsem):
    wid = lax.axis_index("s") * NC + lax.axis_index("c")
    nwh = NW // 2
    b0 = (wid % nwh) * BPW
    jf = wid // nwh

    pltpu.sync_copy(pos_hbm.at[pl.ds(b0, BPW)], pos_v)

    tpc = CW // LANES
    gathers = []
    for k in range(CH):
        for u in range(tpc):
            t = k * tpc + u
            p = pos_v[pl.ds(t * LANES, LANES)]
            blane = b0 + t * LANES + lax.iota(jnp.int32, LANES)
            pm = p - 1
            pm = jnp.where(pm < 0, pm + S, pm)
            row = pm + (p - pm) * jf
            idx_v[k, pl.ds(u * LANES, LANES)] = row * B + blane
        gathers.append(
            pltpu.async_copy(
                stack_hbm.at[idx_v.at[k]],
                rows_v.at[pl.ds(k * CW, CW)],
                gsem,
            )
        )

    writes = []
    for k in range(CH):
        gathers[k].wait()
        writes.append(
            pltpu.async_copy(
                rows_v.at[pl.ds(k * CW, CW)],
                out_hbm.at[pl.ds(wid * BPW + k * CW, CW)],
                wsem,
            )
        )
    for w in writes:
        w.wait()


def kernel(input, op, hidden_stack, pos):
    del input, op
    stack_flat = hidden_stack.reshape(S * B, H)
    out = _stack_gather(stack_flat, pos)
    return out.reshape(2, B, H)
